# Initial kernel scaffold; baseline (speedup 1.0000x reference)
import jax
import jax.numpy as jnp
from jax.experimental import pallas as pl


def kernel(x, edge_index, adj_values, W, b):
    raise NotImplementedError("write your pallas kernel here")


# speedup vs baseline: 4.3969x; 4.3969x over previous
import functools

import jax
import jax.numpy as jnp
from jax import lax
from jax.experimental import pallas as pl
from jax.experimental.pallas import tpu as pltpu
from jax.experimental.pallas import tpu_sc as plsc

N = 10000
E = 320000
D = 128

NC = 2
NS = 16
LANES = 16
NW = NC * NS
CHUNK = 128
CHUNKS_PER_W = (E + NW * CHUNK - 1) // (NW * CHUNK)
EP = NW * CHUNKS_PER_W * CHUNK
NPAD = 10240
ROWS_PER_TILE = NPAD // NS

_mesh = plsc.VectorSubcoreMesh(
    core_axis_name="c", subcore_axis_name="s", num_cores=NC, num_subcores=NS
)


@functools.partial(
    pl.kernel,
    out_type=jax.ShapeDtypeStruct((NC, NPAD, D), jnp.float32),
    mesh=_mesh,
    scratch_types=[
        pltpu.VMEM((CHUNKS_PER_W, CHUNK), jnp.int32),
        pltpu.VMEM((CHUNKS_PER_W, CHUNK), jnp.int32),
        pltpu.VMEM((CHUNKS_PER_W, CHUNK), jnp.float32),
        pltpu.VMEM((CHUNK, D), jnp.float32),
        pltpu.VMEM_SHARED((NPAD, D), jnp.float32),
        pltpu.SemaphoreType.DMA,
    ],
)
def _sc_aggregate(x_hbm, src_hbm, dst_hbm, vals_hbm, z_hbm, out_hbm,
                  src_v, dst_v, vals_v, rows_v, acc, sem):
    cid = lax.axis_index("c")
    sid = lax.axis_index("s")
    wid = sid * NC + cid

    row0 = sid * ROWS_PER_TILE
    pltpu.sync_copy(z_hbm.at[pl.ds(row0, ROWS_PER_TILE)],
                    acc.at[pl.ds(row0, ROWS_PER_TILE)])

    pltpu.sync_copy(src_hbm.at[wid], src_v)
    pltpu.sync_copy(dst_hbm.at[wid], dst_v)
    pltpu.sync_copy(vals_hbm.at[wid], vals_v)
    plsc.subcore_barrier()

    def chunk_body(c, _):
        pltpu.async_copy(x_hbm.at[src_v.at[c]], rows_v, sem).wait()

        def group_body(g, _):
            base = g * LANES
            v16 = vals_v[c, pl.ds(base, LANES)]
            for j in range(LANES):
                e = base + j
                val = jnp.full((LANES,), v16[j], jnp.float32)
                for s in range(D // LANES):
                    cs = LANES * s
                    rows_v[e, pl.ds(cs, LANES)] = rows_v[e, pl.ds(cs, LANES)] * val
            return 0

        lax.fori_loop(0, CHUNK // LANES, group_body, 0)

        pltpu.sync_copy(rows_v, acc.at[dst_v.at[c]], add=True)
        return 0

    lax.fori_loop(0, CHUNKS_PER_W, chunk_body, 0)

    plsc.subcore_barrier()
    pltpu.sync_copy(acc.at[pl.ds(row0, ROWS_PER_TILE)],
                    out_hbm.at[cid, pl.ds(row0, ROWS_PER_TILE)])


def _tc_linear_body(p_ref, w_ref, b_ref, o_ref):
    agg = p_ref[0] + p_ref[1]
    o_ref[...] = jnp.dot(agg, w_ref[...],
                         preferred_element_type=jnp.float32,
                         precision=lax.Precision.HIGHEST) + b_ref[...]


_TC_BLOCK = 2000

_tc_linear = pl.pallas_call(
    _tc_linear_body,
    grid=(N // _TC_BLOCK,),
    in_specs=[
        pl.BlockSpec((NC, _TC_BLOCK, D), lambda i: (0, i, 0)),
        pl.BlockSpec((D, D), lambda i: (0, 0)),
        pl.BlockSpec((1, D), lambda i: (0, 0)),
    ],
    out_specs=pl.BlockSpec((_TC_BLOCK, D), lambda i: (i, 0)),
    out_shape=jax.ShapeDtypeStruct((N, D), jnp.float32),
)


def kernel(x, edge_index, adj_values, W, b):
    dst = edge_index[0].astype(jnp.int32)
    src = edge_index[1].astype(jnp.int32)
    pad = EP - E
    src_p = jnp.concatenate([src, jnp.zeros((pad,), jnp.int32)]).reshape(NW, CHUNKS_PER_W, CHUNK)
    dst_p = jnp.concatenate([dst, jnp.zeros((pad,), jnp.int32)]).reshape(NW, CHUNKS_PER_W, CHUNK)
    vals_p = jnp.concatenate([adj_values, jnp.zeros((pad,), jnp.float32)]).reshape(NW, CHUNKS_PER_W, CHUNK)
    zeros = jnp.zeros((NPAD, D), jnp.float32)
    partials = _sc_aggregate(x, src_p, dst_p, vals_p, zeros)
    return _tc_linear(partials, W.T, b.reshape(1, D))
